# baseline (speedup 1.0000x reference)
import functools

import jax
import jax.numpy as jnp
from jax import lax
from jax.experimental import pallas as pl
from jax.experimental.pallas import tpu as pltpu
from jax.experimental.pallas import tpu_sc as plsc

BATCH = 16384
OUT_DIM = 64
ID_DIM = 32
TEXT_DIM = 128

_NC = 2
_NS = 16
_NW = _NC * _NS
_BPW = BATCH // _NW
_CH = 256
_K = 16


def _sc_gather_one(ids_hbm, emb_hbm, out_hbm, id_s, rows, sem):
    wid = lax.axis_index("s") * _NC + lax.axis_index("c")
    base = wid * _BPW
    pltpu.sync_copy(ids_hbm.at[pl.ds(base, _BPW)], id_s)

    for c in range(_BPW // _CH):

        def group_body(g, c=c):
            off = c * _CH + g * _K
            vec = id_s[pl.ds(off, _K)]
            cps = []
            for j in range(_K):
                rid = vec[j]
                cps.append(pltpu.async_copy(
                    emb_hbm.at[pl.ds(rid, 1)],
                    rows.at[pl.ds(g * _K + j, 1)], sem))
            for cp in cps:
                cp.wait()

        pl.loop(0, _CH // _K)(group_body)
        pltpu.sync_copy(rows, out_hbm.at[pl.ds(base + c * _CH, _CH)])


def _sc_gather_stream(ids_hbm, emb_hbm, out_hbm, idx_v, rows_v, sem):
    wid = lax.axis_index("s") * _NC + lax.axis_index("c")
    base = wid * _BPW
    pltpu.sync_copy(ids_hbm.at[pl.ds(base, _BPW)], idx_v)
    pltpu.async_copy(emb_hbm.at[idx_v], rows_v, sem).wait()
    pltpu.sync_copy(rows_v, out_hbm.at[pl.ds(base, _BPW)])


@functools.cache
def _gather_call(dim, linear=False):
    if linear:
        return pl.kernel(
            _sc_gather_stream,
            mesh=plsc.VectorSubcoreMesh(core_axis_name="c",
                                        subcore_axis_name="s"),
            out_type=jax.ShapeDtypeStruct((BATCH, dim), jnp.float32),
            scratch_types=[
                pltpu.VMEM((_BPW,), jnp.int32),
                pltpu.VMEM((_BPW, dim), jnp.float32),
                pltpu.SemaphoreType.DMA,
            ],
            compiler_params=pltpu.CompilerParams(use_tc_tiling_on_sc=False),
        )
    return pl.kernel(
        _sc_gather_one,
        mesh=plsc.VectorSubcoreMesh(core_axis_name="c", subcore_axis_name="s"),
        out_type=jax.ShapeDtypeStruct((BATCH, dim), jnp.float32),
        scratch_types=[
            pltpu.VMEM((_BPW,), jnp.int32),
            pltpu.VMEM((_CH, dim), jnp.float32),
            pltpu.SemaphoreType.DMA,
        ],
    )


_TC_ROWS = 512
_N_BLOCKS = BATCH // _TC_ROWS


def _tc_text(x_ref, w_ref, b_ref, out_ref):
    t = jnp.dot(x_ref[...], w_ref[...], preferred_element_type=jnp.float32)
    out_ref[...] = t + b_ref[...]


def _text_proj(x, W_text, b2):
    return pl.pallas_call(
        _tc_text,
        grid=(_N_BLOCKS,),
        in_specs=[
            pl.BlockSpec((_TC_ROWS, TEXT_DIM), lambda i: (i, 0)),
            pl.BlockSpec((TEXT_DIM, ID_DIM), lambda i: (0, 0)),
            pl.BlockSpec((1, ID_DIM), lambda i: (0, 0)),
        ],
        out_specs=pl.BlockSpec((_TC_ROWS, ID_DIM), lambda i: (i, 0)),
        out_shape=jax.ShapeDtypeStruct((BATCH, ID_DIM), jnp.float32),
    )(x, W_text, b2)


def _tc_combine(t_ref, u_ref, id_ref, out_ref):
    s = jnp.sum(u_ref[:, :ID_DIM] * id_ref[...], axis=1)
    s = s + jnp.sum(u_ref[:, ID_DIM:] * t_ref[...], axis=1)
    out_ref[...] = jax.nn.sigmoid(s)


def _combine(t, u_gath, i_gath):
    return pl.pallas_call(
        _tc_combine,
        grid=(_N_BLOCKS,),
        in_specs=[
            pl.BlockSpec((_TC_ROWS, ID_DIM), lambda i: (i, 0)),
            pl.BlockSpec((_TC_ROWS, OUT_DIM), lambda i: (i, 0)),
            pl.BlockSpec((_TC_ROWS, ID_DIM), lambda i: (i, 0)),
        ],
        out_specs=pl.BlockSpec((_TC_ROWS,), lambda i: (i,)),
        out_shape=jax.ShapeDtypeStruct((BATCH,), jnp.float32),
    )(t, u_gath, i_gath)


_TR_BLK = 2048


def _tc_transpose(x_ref, o_ref):
    o_ref[...] = x_ref[...].T


def _transpose_table(table_t, dim):
    n = table_t.shape[1]
    return pl.pallas_call(
        _tc_transpose,
        grid=(pl.cdiv(n, _TR_BLK),),
        in_specs=[pl.BlockSpec((dim, _TR_BLK), lambda i: (0, i))],
        out_specs=pl.BlockSpec((_TR_BLK, dim), lambda i: (i, 0)),
        out_shape=jax.ShapeDtypeStruct((n, dim), jnp.float32),
    )(table_t)


def kernel(user_ids, item_ids, item_text_feats, user_emb, item_id_emb,
           W_text, b_text):
    t = _text_proj(item_text_feats, W_text, b_text.reshape(1, ID_DIM))
    u_rm = _transpose_table(user_emb.T, OUT_DIM)
    i_rm = _transpose_table(item_id_emb.T, ID_DIM)
    u_gath = _gather_call(OUT_DIM)(user_ids, u_rm)
    i_gath = _gather_call(ID_DIM)(item_ids, i_rm)
    return _combine(t, u_gath, i_gath)


# speedup vs baseline: 1.3930x; 1.3930x over previous
import functools

import jax
import jax.numpy as jnp
from jax import lax
from jax.experimental import pallas as pl
from jax.experimental.pallas import tpu as pltpu
from jax.experimental.pallas import tpu_sc as plsc

BATCH = 16384
OUT_DIM = 64
ID_DIM = 32
TEXT_DIM = 128

_NC = 2
_NS = 16
_NW = _NC * _NS
_BPW = BATCH // _NW
_CH = 256
_K = 16


def _sc_gather_one(ids_hbm, emb_hbm, out_hbm, id_s, rows, sem):
    wid = lax.axis_index("s") * _NC + lax.axis_index("c")
    base = wid * _BPW
    pltpu.sync_copy(ids_hbm.at[pl.ds(base, _BPW)], id_s)

    for c in range(_BPW // _CH):

        def group_body(g, c=c):
            off = c * _CH + g * _K
            vec = id_s[pl.ds(off, _K)]
            cps = []
            for j in range(_K):
                rid = vec[j]
                cps.append(pltpu.async_copy(
                    emb_hbm.at[pl.ds(rid, 1)],
                    rows.at[pl.ds(g * _K + j, 1)], sem))
            for cp in cps:
                cp.wait()

        pl.loop(0, _CH // _K)(group_body)
        pltpu.sync_copy(rows, out_hbm.at[pl.ds(base + c * _CH, _CH)])


@functools.cache
def _gather_call(dim):
    return pl.kernel(
        _sc_gather_one,
        mesh=plsc.VectorSubcoreMesh(core_axis_name="c", subcore_axis_name="s"),
        out_type=jax.ShapeDtypeStruct((BATCH, dim), jnp.float32),
        scratch_types=[
            pltpu.VMEM((_BPW,), jnp.int32),
            pltpu.VMEM((_CH, dim), jnp.float32),
            pltpu.SemaphoreType.DMA,
        ],
    )


_TC_ROWS = 512
_N_BLOCKS = BATCH // _TC_ROWS


def _tc_text(x_ref, w_ref, b_ref, out_ref):
    t = jnp.dot(x_ref[...], w_ref[...], preferred_element_type=jnp.float32)
    out_ref[...] = t + b_ref[...]


def _text_proj(x, W_text, b2):
    return pl.pallas_call(
        _tc_text,
        grid=(_N_BLOCKS,),
        in_specs=[
            pl.BlockSpec((_TC_ROWS, TEXT_DIM), lambda i: (i, 0)),
            pl.BlockSpec((TEXT_DIM, ID_DIM), lambda i: (0, 0)),
            pl.BlockSpec((1, ID_DIM), lambda i: (0, 0)),
        ],
        out_specs=pl.BlockSpec((_TC_ROWS, ID_DIM), lambda i: (i, 0)),
        out_shape=jax.ShapeDtypeStruct((BATCH, ID_DIM), jnp.float32),
    )(x, W_text, b2)


def _tc_combine(t_ref, u_ref, id_ref, out_ref):
    s = jnp.sum(u_ref[:, :ID_DIM] * id_ref[...], axis=1)
    s = s + jnp.sum(u_ref[:, ID_DIM:] * t_ref[...], axis=1)
    out_ref[...] = jax.nn.sigmoid(s)


def _combine(t, u_gath, i_gath):
    return pl.pallas_call(
        _tc_combine,
        grid=(_N_BLOCKS,),
        in_specs=[
            pl.BlockSpec((_TC_ROWS, ID_DIM), lambda i: (i, 0)),
            pl.BlockSpec((_TC_ROWS, OUT_DIM), lambda i: (i, 0)),
            pl.BlockSpec((_TC_ROWS, ID_DIM), lambda i: (i, 0)),
        ],
        out_specs=pl.BlockSpec((_TC_ROWS,), lambda i: (i,)),
        out_shape=jax.ShapeDtypeStruct((BATCH,), jnp.float32),
    )(t, u_gath, i_gath)


def kernel(user_ids, item_ids, item_text_feats, user_emb, item_id_emb,
           W_text, b_text):
    t = _text_proj(item_text_feats, W_text, b_text.reshape(1, ID_DIM))
    u_gath = _gather_call(OUT_DIM)(user_ids, user_emb)
    i_gath = _gather_call(ID_DIM)(item_ids, item_id_emb)
    return _combine(t, u_gath, i_gath)
